# baseline (speedup 1.0000x reference)
import functools

import jax
import jax.numpy as jnp
from jax import lax
from jax.experimental import pallas as pl
from jax.experimental.pallas import tpu as pltpu
from jax.experimental.pallas import tpu_sc as plsc

_NC, _NS, _LANES = 2, 16, 16
_NW = _NC * _NS

_RBLK = 32
_SBLK = 512
_S_SC = 1024


def _add_body(x_ref, e_ref, o_ref):
    o_ref[...] = x_ref[...] + e_ref[...]


def _tc_add(B, S, D, s_tc, dtype):
    return pl.pallas_call(
        _add_body,
        grid=(s_tc // _SBLK,),
        in_specs=[
            pl.BlockSpec((B, _SBLK, D), lambda i: (0, i, 0)),
            pl.BlockSpec((_SBLK, D), lambda i: (i, 0)),
        ],
        out_specs=pl.BlockSpec((B, _SBLK, D), lambda i: (0, i, 0)),
        out_shape=jax.ShapeDtypeStruct((B, S, D), dtype),
    )


def _sc_add(B, S, D, s1, s2):
    mesh = plsc.VectorSubcoreMesh(core_axis_name="c", subcore_axis_name="s")
    s_per_w = s2 // _NW
    n_blk = s_per_w // _RBLK
    n_chunk = D // _LANES

    @functools.partial(
        pl.kernel,
        out_type=jax.ShapeDtypeStruct((B * s2, D), jnp.float32),
        mesh=mesh,
        scratch_types=[
            pltpu.VMEM((_RBLK, D), jnp.float32),
            pltpu.VMEM((_RBLK, D), jnp.float32),
        ],
    )
    def k(in_hbm, emb_hbm, out_hbm, emb_buf, in_buf):
        wid = lax.axis_index("s") * _NC + lax.axis_index("c")
        r_base = wid * s_per_w

        def blk_body(blk, carry):
            r0 = r_base + blk * _RBLK
            pltpu.sync_copy(emb_hbm.at[pl.ds(s1 + r0, _RBLK), :], emb_buf)

            def b_body(b, carry2):
                pltpu.sync_copy(
                    in_hbm.at[pl.ds(b * S + s1 + r0, _RBLK), :], in_buf)

                def row_body(i, c3):
                    for j in range(n_chunk):
                        e = emb_buf[i, pl.ds(j * _LANES, _LANES)]
                        plsc.addupdate(in_buf.at[i, pl.ds(j * _LANES, _LANES)], e)
                    return c3

                lax.fori_loop(0, _RBLK, row_body, 0, unroll=False)
                pltpu.sync_copy(in_buf, out_hbm.at[pl.ds(b * s2 + r0, _RBLK), :])
                return carry2

            lax.fori_loop(0, B, b_body, 0, unroll=False)
            return carry

        lax.fori_loop(0, n_blk, blk_body, 0, unroll=False)

    return k


def kernel(inputs, embeddings):
    B, S, D = inputs.shape
    pos = embeddings[:S]
    s2 = _S_SC
    s1 = S - s2
    sc_out = _sc_add(B, S, D, s1, s2)(inputs.reshape(B * S, D), pos)
    tc_out = _tc_add(B, S, D, s1, inputs.dtype)(inputs, pos)
    return lax.dynamic_update_slice(
        tc_out, sc_out.reshape(B, s2, D), (0, s1, 0))


# speedup vs baseline: 1.2832x; 1.2832x over previous
import jax
import jax.numpy as jnp
from jax.experimental import pallas as pl


def _add_body(x_ref, e_ref, o_ref):
    o_ref[...] = x_ref[...] + e_ref[...]


def kernel(inputs, embeddings):
    B, S, D = inputs.shape
    pos = embeddings[:S]
    SBLK = 512
    BBLK = 2
    grid = (S // SBLK, B // BBLK)
    return pl.pallas_call(
        _add_body,
        grid=grid,
        in_specs=[
            pl.BlockSpec((BBLK, SBLK, D), lambda i, j: (j, i, 0)),
            pl.BlockSpec((SBLK, D), lambda i, j: (i, 0)),
        ],
        out_specs=pl.BlockSpec((BBLK, SBLK, D), lambda i, j: (j, i, 0)),
        out_shape=jax.ShapeDtypeStruct((B, S, D), inputs.dtype),
    )(inputs, pos)


# speedup vs baseline: 1.3148x; 1.0246x over previous
import jax
import jax.numpy as jnp
from jax import lax
from jax.experimental import pallas as pl
from jax.experimental.pallas import tpu as pltpu

_SBLK = 512
_K = 3
_KO = 3


def _body(in_hbm, emb_hbm, out_hbm, in_buf, emb_buf, out_buf,
          in_sem, emb_sem, out_sem):
    i = pl.program_id(0)
    n = pl.num_programs(0)

    def in_copy(idx, slot):
        return pltpu.make_async_copy(
            in_hbm.at[:, pl.ds(idx * _SBLK, _SBLK), :],
            in_buf.at[slot], in_sem.at[slot])

    def emb_copy(idx, slot):
        return pltpu.make_async_copy(
            emb_hbm.at[pl.ds(idx * _SBLK, _SBLK), :],
            emb_buf.at[slot], emb_sem.at[slot])

    def out_copy(idx, slot):
        return pltpu.make_async_copy(
            out_buf.at[slot], out_hbm.at[:, pl.ds(idx * _SBLK, _SBLK), :],
            out_sem.at[slot])

    @pl.when(i == 0)
    def _():
        for k in range(_K - 1):
            in_copy(k, k).start()
            emb_copy(k, k).start()

    islot = lax.rem(i, _K)
    oslot = lax.rem(i, _KO)

    @pl.when(i + _K - 1 < n)
    def _():
        nslot = lax.rem(i + _K - 1, _K)
        in_copy(i + _K - 1, nslot).start()
        emb_copy(i + _K - 1, nslot).start()

    in_copy(i, islot).wait()
    emb_copy(i, islot).wait()

    @pl.when(i >= _KO)
    def _():
        out_copy(i - _KO, oslot).wait()

    out_buf[oslot] = in_buf[islot] + emb_buf[islot][None]
    out_copy(i, oslot).start()

    @pl.when(i == n - 1)
    def _():
        for k in range(_KO):
            idx = n - _KO + k
            out_copy(idx, lax.rem(idx, _KO)).wait()


def kernel(inputs, embeddings):
    B, S, D = inputs.shape
    pos = embeddings[:S]
    return pl.pallas_call(
        _body,
        grid=(S // _SBLK,),
        in_specs=[
            pl.BlockSpec(memory_space=pl.ANY),
            pl.BlockSpec(memory_space=pl.ANY),
        ],
        out_specs=pl.BlockSpec(memory_space=pl.ANY),
        out_shape=jax.ShapeDtypeStruct((B, S, D), inputs.dtype),
        scratch_shapes=[
            pltpu.VMEM((_K, B, _SBLK, D), inputs.dtype),
            pltpu.VMEM((_K, _SBLK, D), inputs.dtype),
            pltpu.VMEM((_KO, B, _SBLK, D), inputs.dtype),
            pltpu.SemaphoreType.DMA((_K,)),
            pltpu.SemaphoreType.DMA((_K,)),
            pltpu.SemaphoreType.DMA((_KO,)),
        ],
    )(inputs, pos)
